# baseline (speedup 1.0000x reference)
import functools

import jax
import jax.numpy as jnp
from jax import lax
from jax.experimental import pallas as pl
from jax.experimental.pallas import tpu as pltpu
from jax.experimental.pallas import tpu_sc as plsc

_B = 16384
_F = 26
_VOCAB = 100000
_D = 32
_N = _B * _F

_NC = 2
_NS = 16
_NW = _NC * _NS
_ROWS_PER_W = _N // _NW
_CHUNK = 128
_NCHUNK = _ROWS_PER_W // _CHUNK


@functools.partial(
    pl.kernel,
    out_type=jax.ShapeDtypeStruct((_N, _D), jnp.float32),
    mesh=plsc.VectorSubcoreMesh(core_axis_name="c", subcore_axis_name="s"),
    scratch_types=[
        pltpu.VMEM((_CHUNK,), jnp.int32),
        pltpu.VMEM((_CHUNK, _D), jnp.float32),
        pltpu.SemaphoreType.DMA,
    ],
    compiler_params=pltpu.CompilerParams(use_tc_tiling_on_sc=False),
)
def _embed_kernel(x_hbm, table_hbm, out_hbm, idx_v, rows_v, sem):
    wid = lax.axis_index("s") * _NC + lax.axis_index("c")
    base = wid * _ROWS_PER_W
    lanes = lax.iota(jnp.int32, 16)

    def chunk_body(c, carry):
        start = base + c * _CHUNK
        pltpu.sync_copy(x_hbm.at[pl.ds(start, _CHUNK)], idx_v)

        def vec_body(j, carry2):
            pos = start + j * 16 + lanes
            off = (pos % _F) * _VOCAB
            idx_v[pl.ds(j * 16, 16)] = idx_v[pl.ds(j * 16, 16)] + off
            return carry2

        lax.fori_loop(0, _CHUNK // 16, vec_body, 0)
        pltpu.async_copy(table_hbm.at[idx_v], rows_v, sem).wait()
        pltpu.sync_copy(rows_v, out_hbm.at[pl.ds(start, _CHUNK)])
        return carry

    lax.fori_loop(0, _NCHUNK, chunk_body, 0)


def kernel(x, tables):
    x_flat = x.reshape(_N)
    table2d = tables.reshape(_F * _VOCAB, _D)
    out = _embed_kernel(x_flat, table2d)
    return out.reshape(_B, _F, _D)


# speedup vs baseline: 1.0851x; 1.0851x over previous
import functools

import jax
import jax.numpy as jnp
from jax import lax
from jax.experimental import pallas as pl
from jax.experimental.pallas import tpu as pltpu
from jax.experimental.pallas import tpu_sc as plsc

_B = 16384
_F = 26
_VOCAB = 100000
_D = 32
_N = _B * _F

_NC = 2
_NS = 16
_NW = _NC * _NS
_ROWS_PER_W = _N // _NW
_BLK = 1664
_NBLK = _ROWS_PER_W // _BLK


@functools.partial(
    pl.kernel,
    out_type=jax.ShapeDtypeStruct((_N, _D), jnp.float32),
    mesh=plsc.VectorSubcoreMesh(core_axis_name="c", subcore_axis_name="s"),
    scratch_types=[
        pltpu.VMEM((_NBLK, _BLK), jnp.int32),
        pltpu.VMEM((_BLK, _D), jnp.float32),
        pltpu.VMEM((_BLK, _D), jnp.float32),
        pltpu.SemaphoreType.DMA,
        pltpu.SemaphoreType.DMA,
        pltpu.SemaphoreType.DMA,
        pltpu.SemaphoreType.DMA,
    ],
    compiler_params=pltpu.CompilerParams(use_tc_tiling_on_sc=False),
)
def _embed_kernel(x_hbm, table_hbm, out_hbm, idx_v, rows0, rows1, g0, g1, w0, w1):
    wid = lax.axis_index("s") * _NC + lax.axis_index("c")
    base = wid * _ROWS_PER_W
    lanes = lax.iota(jnp.int32, 16)
    bufs = (rows0, rows1)
    gsem = (g0, g1)
    wsem = (w0, w1)

    pltpu.sync_copy(x_hbm.at[pl.ds(wid * _NBLK, _NBLK)], idx_v)

    def rewrite_block(c):
        def vec_body(j, carry):
            pos = base + c * _BLK + j * 16 + lanes
            off = (pos % _F) * _VOCAB
            idx_v[c, pl.ds(j * 16, 16)] = idx_v[c, pl.ds(j * 16, 16)] + off
            return carry

        lax.fori_loop(0, _BLK // 16, vec_body, 0)

    def issue_gather(c):
        return pltpu.async_copy(table_hbm.at[idx_v.at[c]], bufs[c % 2], gsem[c % 2])

    def issue_write(c):
        return pltpu.async_copy(
            bufs[c % 2], out_hbm.at[pl.ds(base + c * _BLK, _BLK)], wsem[c % 2]
        )

    rewrite_block(0)
    gathers = [issue_gather(0)]
    writes = [None, None]
    for c in range(_NBLK):
        p = c % 2
        if c + 1 < _NBLK:
            rewrite_block(c + 1)
            if writes[1 - p] is not None:
                writes[1 - p].wait()
            gathers.append(issue_gather(c + 1))
        gathers[c].wait()
        writes[p] = issue_write(c)
    writes[0].wait()
    writes[1].wait()


def kernel(x, tables):
    x_flat = x.reshape(_N // _BLK, _BLK)
    table2d = tables.reshape(_F * _VOCAB, _D)
    out = _embed_kernel(x_flat, table2d)
    return out.reshape(_B, _F, _D)
